# Initial kernel scaffold; baseline (speedup 1.0000x reference)
import jax
import jax.numpy as jnp
from jax.experimental import pallas as pl


def kernel(edge_attr, edge_index, W0, b0, W1, b1, Wout, bout):
    raise NotImplementedError("write your pallas kernel here")


# speedup vs baseline: 2.1910x; 2.1910x over previous
import functools

import jax
import jax.numpy as jnp
from jax import lax
from jax.experimental import pallas as pl
from jax.experimental.pallas import tpu as pltpu
from jax.experimental.pallas import tpu_sc as plsc

B = 16
N = 1000
E = 32000
D = 128
BE = B * E


TILE = 2000
NPROG = BE // TILE
TPG = E // TILE


def _mlp_body(x_ref, ei_ref, w0_ref, b0_ref, w1_ref, b1_ref, wout_ref,
              bout_ref, val_ref, addr_ref):
    x = x_ref[...]
    h = lax.dot_general(x, w0_ref[...], (((1,), (1,)), ((), ())),
                        preferred_element_type=jnp.float32) + b0_ref[...]
    h = h * lax.logistic(h)
    h = lax.dot_general(h, w1_ref[...], (((1,), (1,)), ((), ())),
                        preferred_element_type=jnp.float32) + b1_ref[...]
    h = h * lax.logistic(h)
    v = lax.dot_general(wout_ref[...], h, (((1,), (1,)), ((), ())),
                        preferred_element_type=jnp.float32) + bout_ref[...]
    val_ref[0] = lax.logistic(v) * 5.0

    g = pl.program_id(0) // TPG
    iv = ei_ref[0, 0, 0]
    jv = ei_ref[0, 1, 0]
    addr_ref[0, 0] = iv * N + jv + g * (N * N)


def _run_mlp(edge_attr, ei_r, W0, b0, W1, b1, Wout, bout):
    return pl.pallas_call(
        _mlp_body,
        grid=(NPROG,),
        in_specs=[
            pl.BlockSpec((TILE, D), lambda t: (t, 0)),
            pl.BlockSpec((1, 2, 1, TILE), lambda t: (t, 0, 0, 0)),
            pl.BlockSpec((D, D), lambda t: (0, 0)),
            pl.BlockSpec((D,), lambda t: (0,)),
            pl.BlockSpec((D, D), lambda t: (0, 0)),
            pl.BlockSpec((D,), lambda t: (0,)),
            pl.BlockSpec((1, D), lambda t: (0, 0)),
            pl.BlockSpec((1,), lambda t: (0,)),
        ],
        out_specs=[
            pl.BlockSpec((1, 1, TILE), lambda t: (t, 0, 0)),
            pl.BlockSpec((1, 1, TILE), lambda t: (t, 0, 0)),
        ],
        out_shape=[
            jax.ShapeDtypeStruct((NPROG, 1, TILE), jnp.float32),
            jax.ShapeDtypeStruct((NPROG, 1, TILE), jnp.int32),
        ],
    )(edge_attr, ei_r, W0, b0, W1, b1, Wout, bout)


NC = 2
NS = 16
GPC = B // NC
WORDS = B * N * N
WORDS_PER_SC = GPC * N * N
WORDS_PER_TILE = WORDS_PER_SC // NS
ZW = 20000
NZ = WORDS_PER_TILE // ZW
RPG = 256
PADR = RPG - E // 128
SROWS = 128
OUT_PAD = PADR * 128
OUT_WORDS = WORDS + OUT_PAD


def _sc_body(addr_hbm, vals_hbm, out_hbm, zbuf, abuf, vbuf, zsem, sem):
    c = lax.axis_index("c")
    s = lax.axis_index("s")

    zeros16 = jnp.zeros((16,), jnp.float32)

    def fill(i, _):
        zbuf[pl.ds(i * 16, 16)] = zeros16
        return 0
    lax.fori_loop(0, ZW // 16, fill, 0)

    zoff = pl.multiple_of(c * WORDS_PER_SC + s * WORDS_PER_TILE, 8)

    def zfire(i, _):
        pltpu.async_copy(zbuf, out_hbm.at[pl.ds(zoff + i * ZW, ZW)], zsem)
        return 0
    lax.fori_loop(0, NZ, zfire, 0)

    t = c * NS + s
    row0 = pl.multiple_of(t * SROWS, 8)
    pltpu.sync_copy(addr_hbm.at[pl.ds(row0, SROWS)], abuf)
    pltpu.sync_copy(vals_hbm.at[pl.ds(row0, SROWS)], vbuf)

    def zdrain(i, _):
        pltpu.make_async_copy(
            zbuf, out_hbm.at[pl.ds(zoff + i * ZW, ZW)], zsem).wait()
        return 0
    lax.fori_loop(0, NZ, zdrain, 0)

    plsc.subcore_barrier()

    def sfire(r, _):
        pltpu.async_copy(vbuf.at[r], out_hbm.at[abuf.at[r]], sem)
        return 0
    lax.fori_loop(0, SROWS, sfire, 0)

    def sdrain(r, _):
        pltpu.make_async_copy(vbuf.at[r], out_hbm.at[abuf.at[r]], sem).wait()
        return 0
    lax.fori_loop(0, SROWS, sdrain, 0)


def _run_scatter(addr2, vals2):
    mesh = plsc.VectorSubcoreMesh(core_axis_name="c", subcore_axis_name="s")
    kern = pl.kernel(
        _sc_body,
        out_type=jax.ShapeDtypeStruct((OUT_WORDS,), jnp.float32),
        mesh=mesh,
        scratch_types=[
            pltpu.VMEM((ZW,), jnp.float32),
            pltpu.VMEM((SROWS, 128), jnp.int32),
            pltpu.VMEM((SROWS, 128), jnp.float32),
            pltpu.SemaphoreType.DMA,
            pltpu.SemaphoreType.DMA,
        ],
    )
    return kern(addr2, vals2)


TS = 256
NBLK = (N + TS - 1) // TS


def _sym_body(a_ref, b_ref, o_ref):
    o_ref[0] = a_ref[0] + b_ref[0].T


def _run_sym(S):
    return pl.pallas_call(
        _sym_body,
        grid=(B, NBLK, NBLK),
        in_specs=[
            pl.BlockSpec((1, TS, TS), lambda b, i, j: (b, i, j)),
            pl.BlockSpec((1, TS, TS), lambda b, i, j: (b, j, i)),
        ],
        out_specs=pl.BlockSpec((1, TS, TS), lambda b, i, j: (b, i, j)),
        out_shape=jax.ShapeDtypeStruct((B, N, N), jnp.float32),
    )(S, S)


@jax.jit
def kernel(edge_attr, edge_index, W0, b0, W1, b1, Wout, bout):
    ei_r = edge_index.reshape(B, 2, TPG, TILE).transpose(0, 2, 1, 3)
    ei_r = ei_r.reshape(NPROG, 2, 1, TILE)
    vals, addr = _run_mlp(edge_attr, ei_r, W0, b0, W1, b1, Wout, bout)
    vals3 = vals.reshape(B, E // 128, 128)
    addr3 = addr.reshape(B, E // 128, 128)
    dump = (WORDS + jnp.arange(PADR * 128, dtype=jnp.int32)).reshape(
        1, PADR, 128)
    addr_p = jnp.concatenate(
        [addr3, jnp.broadcast_to(dump, (B, PADR, 128))], axis=1)
    vals_p = jnp.concatenate(
        [vals3, jnp.zeros((B, PADR, 128), jnp.float32)], axis=1)
    S = _run_scatter(addr_p.reshape(B * RPG, 128),
                     vals_p.reshape(B * RPG, 128))
    return _run_sym(S[:WORDS].reshape(B, N, N))


# speedup vs baseline: 3.1738x; 1.4486x over previous
import functools

import jax
import jax.numpy as jnp
from jax import lax
from jax.experimental import pallas as pl
from jax.experimental.pallas import tpu as pltpu
from jax.experimental.pallas import tpu_sc as plsc

B = 16
N = 1000
E = 32000
D = 128
BE = B * E


TILE = 2000
NPROG = BE // TILE
TPG = E // TILE


def _mlp_body(x_ref, ei_ref, w0_ref, b0_ref, w1_ref, b1_ref, wout_ref,
              bout_ref, val_ref, addr_ref):
    x = x_ref[...]
    h = lax.dot_general(x, w0_ref[...], (((1,), (1,)), ((), ())),
                        preferred_element_type=jnp.float32) + b0_ref[...]
    h = h * lax.logistic(h)
    h = lax.dot_general(h, w1_ref[...], (((1,), (1,)), ((), ())),
                        preferred_element_type=jnp.float32) + b1_ref[...]
    h = h * lax.logistic(h)
    v = lax.dot_general(wout_ref[...], h, (((1,), (1,)), ((), ())),
                        preferred_element_type=jnp.float32) + bout_ref[...]
    val_ref[0] = lax.logistic(v) * 5.0

    iv = ei_ref[0, 0, 0]
    jv = ei_ref[0, 1, 0]
    addr_ref[0, 0] = iv * N + jv


def _run_mlp(edge_attr, ei_r, W0, b0, W1, b1, Wout, bout):
    return pl.pallas_call(
        _mlp_body,
        grid=(NPROG,),
        in_specs=[
            pl.BlockSpec((TILE, D), lambda t: (t, 0)),
            pl.BlockSpec((1, 2, 1, TILE), lambda t: (t, 0, 0, 0)),
            pl.BlockSpec((D, D), lambda t: (0, 0)),
            pl.BlockSpec((D,), lambda t: (0,)),
            pl.BlockSpec((D, D), lambda t: (0, 0)),
            pl.BlockSpec((D,), lambda t: (0,)),
            pl.BlockSpec((1, D), lambda t: (0, 0)),
            pl.BlockSpec((1,), lambda t: (0,)),
        ],
        out_specs=[
            pl.BlockSpec((1, 1, TILE), lambda t: (t, 0, 0)),
            pl.BlockSpec((1, 1, TILE), lambda t: (t, 0, 0)),
        ],
        out_shape=[
            jax.ShapeDtypeStruct((NPROG, 1, TILE), jnp.float32),
            jax.ShapeDtypeStruct((NPROG, 1, TILE), jnp.int32),
        ],
    )(edge_attr, ei_r, W0, b0, W1, b1, Wout, bout)


NC = 2
NS = 16
GPC = B // NC
WORDS = B * N * N
GW = N * N
TR = 62496
TSZ = 62560
CH = 2000
NCH = E // CH


def _sc_body(addr_hbm, vals_hbm, zeros_hbm, out_hbm, grid, abuf, vbuf, sem):
    c = lax.axis_index("c")
    s = lax.axis_index("s")
    lo = s * TR
    zeros16 = jnp.zeros((16,), jnp.float32)

    pltpu.sync_copy(zeros_hbm, grid)

    def round_body(k, _):
        g = c * GPC + k
        ebase = pl.multiple_of(g * E, 8)
        pltpu.sync_copy(addr_hbm.at[pl.ds(ebase, E)], abuf)

        def chunk_body(ch, _):
            pltpu.sync_copy(vals_hbm.at[pl.ds(ebase + ch * CH, CH)], vbuf)

            def grp(q, _):
                a = abuf[pl.ds(ch * CH + q * 16, 16)]
                v = vbuf[pl.ds(q * 16, 16)]
                m = (a >= lo) & (a < lo + TSZ)
                plsc.store_scatter(grid, [a - lo], v, mask=m)
                return 0
            lax.fori_loop(0, CH // 16, grp, 0)
            return 0
        lax.fori_loop(0, NCH, chunk_body, 0)

        oo = pl.multiple_of(g * GW + s * TR, 8)
        pltpu.sync_copy(grid, out_hbm.at[pl.ds(oo, TSZ)])

        def cgrp(q, _):
            a = abuf[pl.ds(q * 16, 16)]
            m = (a >= lo) & (a < lo + TSZ)
            plsc.store_scatter(grid, [a - lo], zeros16, mask=m)
            return 0
        lax.fori_loop(0, E // 16, cgrp, 0)
        return 0

    lax.fori_loop(0, GPC, round_body, 0)


def _run_scatter(addr1, vals1, zeros_w):
    mesh = plsc.VectorSubcoreMesh(core_axis_name="c", subcore_axis_name="s")
    kern = pl.kernel(
        _sc_body,
        out_type=jax.ShapeDtypeStruct((WORDS,), jnp.float32),
        mesh=mesh,
        compiler_params=pltpu.CompilerParams(needs_layout_passes=False),
        scratch_types=[
            pltpu.VMEM((TSZ,), jnp.float32),
            pltpu.VMEM((E,), jnp.int32),
            pltpu.VMEM((CH,), jnp.float32),
            pltpu.SemaphoreType.DMA,
        ],
    )
    return kern(addr1, vals1, zeros_w)


TS = 256
NBLK = (N + TS - 1) // TS


def _sym_body(a_ref, b_ref, o_ref):
    o_ref[0] = a_ref[0] + b_ref[0].T


def _run_sym(S):
    return pl.pallas_call(
        _sym_body,
        grid=(B, NBLK, NBLK),
        in_specs=[
            pl.BlockSpec((1, TS, TS), lambda b, i, j: (b, i, j)),
            pl.BlockSpec((1, TS, TS), lambda b, i, j: (b, j, i)),
        ],
        out_specs=pl.BlockSpec((1, TS, TS), lambda b, i, j: (b, i, j)),
        out_shape=jax.ShapeDtypeStruct((B, N, N), jnp.float32),
    )(S, S)


@jax.jit
def kernel(edge_attr, edge_index, W0, b0, W1, b1, Wout, bout):
    ei_r = edge_index.reshape(B, 2, TPG, TILE).transpose(0, 2, 1, 3)
    ei_r = ei_r.reshape(NPROG, 2, 1, TILE)
    vals, addr = _run_mlp(edge_attr, ei_r, W0, b0, W1, b1, Wout, bout)
    zeros_w = jnp.zeros((TSZ,), jnp.float32)
    S = _run_scatter(addr.reshape(BE), vals.reshape(BE), zeros_w)
    return _run_sym(S.reshape(B, N, N))


# speedup vs baseline: 4.1335x; 1.3024x over previous
import functools

import jax
import jax.numpy as jnp
from jax import lax
from jax.experimental import pallas as pl
from jax.experimental.pallas import tpu as pltpu
from jax.experimental.pallas import tpu_sc as plsc

B = 16
N = 1000
E = 32000
D = 128
BE = B * E


TILE = 2000
NPROG = BE // TILE
TPG = E // TILE


def _addr_body(ei_ref, addr_ref):
    addr_ref[0, 0] = ei_ref[0, 0] * N + ei_ref[0, 1]


def _run_addr(edge_index):
    return pl.pallas_call(
        _addr_body,
        grid=(B,),
        in_specs=[pl.BlockSpec((1, 2, E), lambda b: (b, 0, 0))],
        out_specs=pl.BlockSpec((1, 1, E), lambda b: (b, 0, 0)),
        out_shape=jax.ShapeDtypeStruct((B, 1, E), jnp.int32),
    )(edge_index)


def _mlp_body(x_ref, w0_ref, b0_ref, w1_ref, b1_ref, wout_ref,
              bout_ref, val_ref):
    bf = jnp.bfloat16
    x = x_ref[...].astype(bf)
    h = lax.dot_general(x, w0_ref[...].astype(bf), (((1,), (1,)), ((), ())),
                        preferred_element_type=jnp.float32) + b0_ref[...]
    h = h * lax.logistic(h)
    h = lax.dot_general(h.astype(bf), w1_ref[...].astype(bf),
                        (((1,), (1,)), ((), ())),
                        preferred_element_type=jnp.float32) + b1_ref[...]
    h = h * lax.logistic(h)
    v = lax.dot_general(wout_ref[...], h, (((1,), (1,)), ((), ())),
                        preferred_element_type=jnp.float32) + bout_ref[...]
    val_ref[0] = lax.logistic(v) * 5.0


def _run_mlp(edge_attr, W0, b0, W1, b1, Wout, bout):
    return pl.pallas_call(
        _mlp_body,
        grid=(NPROG,),
        in_specs=[
            pl.BlockSpec((TILE, D), lambda t: (t, 0)),
            pl.BlockSpec((D, D), lambda t: (0, 0)),
            pl.BlockSpec((D,), lambda t: (0,)),
            pl.BlockSpec((D, D), lambda t: (0, 0)),
            pl.BlockSpec((D,), lambda t: (0,)),
            pl.BlockSpec((1, D), lambda t: (0, 0)),
            pl.BlockSpec((1,), lambda t: (0,)),
        ],
        out_specs=pl.BlockSpec((1, 1, TILE), lambda t: (t, 0, 0)),
        out_shape=jax.ShapeDtypeStruct((NPROG, 1, TILE), jnp.float32),
    )(edge_attr, W0, b0, W1, b1, Wout, bout)


NC = 2
NS = 16
GPC = B // NC
WORDS = B * N * N
GW = N * N
TR = 62496
TSZ = 62560
CH = 4000
NCH = E // CH


def _sc_body(addr_hbm, vals_hbm, zeros_hbm, out_hbm,
             grid, abuf0, abuf1, vbuf0, vbuf1, hits, sem0, sem1):
    c = lax.axis_index("c")
    s = lax.axis_index("s")
    lo = s * TR
    zeros16 = jnp.zeros((16,), jnp.float32)
    iota16 = lax.iota(jnp.int32, 16)
    abufs, vbufs, sems = (abuf0, abuf1), (vbuf0, vbuf1), (sem0, sem1)

    pltpu.sync_copy(zeros_hbm, grid)

    def round_body(k, _):
        g = c * GPC + k
        ebase = pl.multiple_of(g * E, 8)

        def fire(ch, slot):
            off = pl.multiple_of(ebase + ch * CH, 8)
            pltpu.async_copy(addr_hbm.at[pl.ds(off, CH)], abufs[slot],
                             sems[slot])
            pltpu.async_copy(vals_hbm.at[pl.ds(off, CH)], vbufs[slot],
                             sems[slot])

        def waitc(slot):
            pltpu.make_async_copy(
                addr_hbm.at[pl.ds(ebase, CH)], abufs[slot], sems[slot]).wait()
            pltpu.make_async_copy(
                vals_hbm.at[pl.ds(ebase, CH)], vbufs[slot], sems[slot]).wait()

        fire(0, 0)
        acc = jnp.int32(0)
        for ch in range(NCH):
            slot = ch % 2
            waitc(slot)
            if ch + 1 < NCH:
                fire(ch + 1, 1 - slot)

            def grp(q, a_cc, slot=slot):
                a = abufs[slot][pl.ds(q * 16, 16)]
                v = vbufs[slot][pl.ds(q * 16, 16)]
                rel = a - lo
                m = plsc.bitcast(rel, jnp.uint32) < jnp.uint32(TSZ)
                plsc.store_scatter(grid, [rel], v, mask=m)
                plsc.store_compressed(hits.at[pl.ds(a_cc, 16)], rel, mask=m)
                cnt = plsc.all_reduce_population_count(m)
                return a_cc + cnt[0]
            acc = lax.fori_loop(0, CH // 16, grp, acc)

        oo = pl.multiple_of(g * GW + s * TR, 8)
        pltpu.sync_copy(grid, out_hbm.at[pl.ds(oo, TSZ)])

        def cgrp(q, _):
            rel = hits[pl.ds(q * 16, 16)]
            m = (q * 16 + iota16) < acc
            plsc.store_scatter(grid, [rel], zeros16, mask=m)
            return 0
        lax.fori_loop(0, (acc + 15) // 16, cgrp, 0)
        return 0

    lax.fori_loop(0, GPC, round_body, 0)


def _run_scatter(addr1, vals1, zeros_w):
    mesh = plsc.VectorSubcoreMesh(core_axis_name="c", subcore_axis_name="s")
    kern = pl.kernel(
        _sc_body,
        out_type=jax.ShapeDtypeStruct((WORDS,), jnp.float32),
        mesh=mesh,
        compiler_params=pltpu.CompilerParams(needs_layout_passes=False),
        scratch_types=[
            pltpu.VMEM((TSZ,), jnp.float32),
            pltpu.VMEM((CH,), jnp.int32),
            pltpu.VMEM((CH,), jnp.int32),
            pltpu.VMEM((CH,), jnp.float32),
            pltpu.VMEM((CH,), jnp.float32),
            pltpu.VMEM((E,), jnp.int32),
            pltpu.SemaphoreType.DMA,
            pltpu.SemaphoreType.DMA,
        ],
    )
    return kern(addr1, vals1, zeros_w)


TS = 256
NBLK = (N + TS - 1) // TS


def _sym_body(a_ref, b_ref, o_ref):
    o_ref[0] = a_ref[0] + b_ref[0].T


def _run_sym(S):
    return pl.pallas_call(
        _sym_body,
        grid=(B, NBLK, NBLK),
        in_specs=[
            pl.BlockSpec((1, TS, TS), lambda b, i, j: (b, i, j)),
            pl.BlockSpec((1, TS, TS), lambda b, i, j: (b, j, i)),
        ],
        out_specs=pl.BlockSpec((1, TS, TS), lambda b, i, j: (b, i, j)),
        out_shape=jax.ShapeDtypeStruct((B, N, N), jnp.float32),
    )(S, S)


@jax.jit
def kernel(edge_attr, edge_index, W0, b0, W1, b1, Wout, bout):
    vals = _run_mlp(edge_attr, W0, b0, W1, b1, Wout, bout)
    addr = _run_addr(edge_index)
    zeros_w = jnp.zeros((TSZ,), jnp.float32)
    S = _run_scatter(addr.reshape(BE), vals.reshape(BE), zeros_w)
    return _run_sym(S.reshape(B, N, N))


# speedup vs baseline: 5.0537x; 1.2226x over previous
import functools

import jax
import jax.numpy as jnp
from jax import lax
from jax.experimental import pallas as pl
from jax.experimental.pallas import tpu as pltpu
from jax.experimental.pallas import tpu_sc as plsc

B = 16
N = 1000
E = 32000
D = 128
BE = B * E


TILE = 2000
NPROG = BE // TILE
TPG = E // TILE


def _addr_body(ei_ref, addr_ref):
    addr_ref[0, 0] = ei_ref[0, 0] * N + ei_ref[0, 1]


def _run_addr(edge_index):
    return pl.pallas_call(
        _addr_body,
        grid=(B,),
        in_specs=[pl.BlockSpec((1, 2, E), lambda b: (b, 0, 0))],
        out_specs=pl.BlockSpec((1, 1, E), lambda b: (b, 0, 0)),
        out_shape=jax.ShapeDtypeStruct((B, 1, E), jnp.int32),
    )(edge_index)


def _mlp_body(x_ref, w0_ref, b0_ref, w1_ref, b1_ref, wout_ref,
              bout_ref, val_ref):
    bf = jnp.bfloat16
    x = x_ref[...].astype(bf)
    h = lax.dot_general(x, w0_ref[...].astype(bf), (((1,), (1,)), ((), ())),
                        preferred_element_type=jnp.float32) + b0_ref[...]
    h = h * lax.logistic(h)
    h = lax.dot_general(h.astype(bf), w1_ref[...].astype(bf),
                        (((1,), (1,)), ((), ())),
                        preferred_element_type=jnp.float32) + b1_ref[...]
    h = h * lax.logistic(h)
    v = lax.dot_general(wout_ref[...], h, (((1,), (1,)), ((), ())),
                        preferred_element_type=jnp.float32) + bout_ref[...]
    val_ref[0] = lax.logistic(v) * 5.0


def _run_mlp(edge_attr, W0, b0, W1, b1, Wout, bout):
    return pl.pallas_call(
        _mlp_body,
        grid=(NPROG,),
        in_specs=[
            pl.BlockSpec((TILE, D), lambda t: (t, 0)),
            pl.BlockSpec((D, D), lambda t: (0, 0)),
            pl.BlockSpec((D,), lambda t: (0,)),
            pl.BlockSpec((D, D), lambda t: (0, 0)),
            pl.BlockSpec((D,), lambda t: (0,)),
            pl.BlockSpec((1, D), lambda t: (0, 0)),
            pl.BlockSpec((1,), lambda t: (0,)),
        ],
        out_specs=pl.BlockSpec((1, 1, TILE), lambda t: (t, 0, 0)),
        out_shape=jax.ShapeDtypeStruct((NPROG, 1, TILE), jnp.float32),
    )(edge_attr, W0, b0, W1, b1, Wout, bout)


NC = 2
NS = 16
GPC = B // NC
WORDS = B * N * N
GW = N * N
TR = 62496
TSZ = 62560
CH = 4000
NCH = E // CH


def _sc_body(addr_hbm, vals_hbm, zeros_hbm, out_hbm,
             grid, abuf0, abuf1, vbuf0, vbuf1, hits, sem0, sem1):
    c = lax.axis_index("c")
    s = lax.axis_index("s")
    lo = s * TR
    zeros16 = jnp.zeros((16,), jnp.float32)
    iota16 = lax.iota(jnp.int32, 16)
    abufs, vbufs, sems = (abuf0, abuf1), (vbuf0, vbuf1), (sem0, sem1)

    pltpu.sync_copy(zeros_hbm, grid)

    def round_body(k, _):
        g = c * GPC + k
        ebase = pl.multiple_of(g * E, 8)

        def fire(ch, slot):
            off = pl.multiple_of(ebase + ch * CH, 8)
            pltpu.async_copy(addr_hbm.at[pl.ds(off, CH)], abufs[slot],
                             sems[slot])
            pltpu.async_copy(vals_hbm.at[pl.ds(off, CH)], vbufs[slot],
                             sems[slot])

        def waitc(slot):
            pltpu.make_async_copy(
                addr_hbm.at[pl.ds(ebase, CH)], abufs[slot], sems[slot]).wait()
            pltpu.make_async_copy(
                vals_hbm.at[pl.ds(ebase, CH)], vbufs[slot], sems[slot]).wait()

        fire(0, 0)
        acc = jnp.int32(0)
        for ch in range(NCH):
            slot = ch % 2
            waitc(slot)
            if ch + 1 < NCH:
                fire(ch + 1, 1 - slot)

            def grp(q, a_cc, slot=slot):
                a = abufs[slot][pl.ds(q * 16, 16)]
                v = vbufs[slot][pl.ds(q * 16, 16)]
                rel = a - lo
                m = plsc.bitcast(rel, jnp.uint32) < jnp.uint32(TSZ)
                plsc.store_scatter(grid, [rel], v, mask=m)
                plsc.store_compressed(hits.at[pl.ds(a_cc, 16)], rel, mask=m)
                cnt = plsc.all_reduce_population_count(m)
                return a_cc + cnt[0]
            acc = lax.fori_loop(0, CH // 16, grp, acc)

        oo = pl.multiple_of(g * GW + s * TR, 8)
        pltpu.sync_copy(grid, out_hbm.at[pl.ds(oo, TSZ)])

        def cgrp(q, _):
            rel = hits[pl.ds(q * 16, 16)]
            m = (q * 16 + iota16) < acc
            plsc.store_scatter(grid, [rel], zeros16, mask=m)
            return 0
        lax.fori_loop(0, (acc + 15) // 16, cgrp, 0)
        return 0

    lax.fori_loop(0, GPC, round_body, 0)


def _run_scatter(addr1, vals1, zeros_w):
    mesh = plsc.VectorSubcoreMesh(core_axis_name="c", subcore_axis_name="s")
    kern = pl.kernel(
        _sc_body,
        out_type=jax.ShapeDtypeStruct((WORDS,), jnp.float32),
        mesh=mesh,
        compiler_params=pltpu.CompilerParams(needs_layout_passes=False),
        scratch_types=[
            pltpu.VMEM((TSZ,), jnp.float32),
            pltpu.VMEM((CH,), jnp.int32),
            pltpu.VMEM((CH,), jnp.int32),
            pltpu.VMEM((CH,), jnp.float32),
            pltpu.VMEM((CH,), jnp.float32),
            pltpu.VMEM((E,), jnp.int32),
            pltpu.SemaphoreType.DMA,
            pltpu.SemaphoreType.DMA,
        ],
    )
    return kern(addr1, vals1, zeros_w)


def _sym_body(a_ref, o_ref):
    x = a_ref[...]
    o_ref[0] = x + x.T


def _run_sym(S2):
    return pl.pallas_call(
        _sym_body,
        grid=(B,),
        in_specs=[pl.BlockSpec((N, N), lambda b: (b, 0))],
        out_specs=pl.BlockSpec((1, N, N), lambda b: (b, 0, 0)),
        out_shape=jax.ShapeDtypeStruct((B, N, N), jnp.float32),
    )(S2)


@jax.jit
def kernel(edge_attr, edge_index, W0, b0, W1, b1, Wout, bout):
    vals = _run_mlp(edge_attr, W0, b0, W1, b1, Wout, bout)
    addr = _run_addr(edge_index)
    zeros_w = jnp.zeros((TSZ,), jnp.float32)
    S = _run_scatter(addr.reshape(BE), vals.reshape(BE), zeros_w)
    return _run_sym(S.reshape(B * N, N))


# speedup vs baseline: 6.5464x; 1.2954x over previous
import functools

import jax
import jax.numpy as jnp
from jax import lax
from jax.experimental import pallas as pl
from jax.experimental.pallas import tpu as pltpu
from jax.experimental.pallas import tpu_sc as plsc

B = 16
N = 1000
E = 32000
D = 128
BE = B * E


TILE = 16000
NPROG = BE // TILE
TPG = E // TILE


def _addr_body(ei_ref, addr_ref):
    addr_ref[0, 0] = ei_ref[0, 0] * N + ei_ref[0, 1]


def _run_addr(edge_index):
    return pl.pallas_call(
        _addr_body,
        grid=(B,),
        in_specs=[pl.BlockSpec((1, 2, E), lambda b: (b, 0, 0))],
        out_specs=pl.BlockSpec((1, 1, E), lambda b: (b, 0, 0)),
        out_shape=jax.ShapeDtypeStruct((B, 1, E), jnp.int32),
    )(edge_index)


def _mlp_body(x_ref, w0_ref, b0_ref, w1_ref, b1_ref, wout_ref,
              bout_ref, val_ref):
    bf = jnp.bfloat16
    w0 = w0_ref[...].astype(bf)
    w1 = w1_ref[...].astype(bf)

    def head(x):
        a1 = lax.dot_general(x, w0, (((1,), (1,)), ((), ())),
                             preferred_element_type=jnp.float32) + b0_ref[...]
        h1 = a1.astype(bf)
        h1 = h1 * lax.logistic(h1)
        a2 = lax.dot_general(h1, w1, (((1,), (1,)), ((), ())),
                             preferred_element_type=jnp.float32) + b1_ref[...]
        h2 = a2.astype(bf)
        h = (h2 * lax.logistic(h2)).astype(jnp.float32)
        return lax.dot_general(wout_ref[...], h, (((1,), (1,)), ((), ())),
                               preferred_element_type=jnp.float32)

    x = x_ref[...].astype(bf)
    v = head(x) + bout_ref[...]
    val_ref[0] = lax.logistic(v) * 5.0


def _run_mlp(edge_attr, W0, b0, W1, b1, Wout, bout):
    return pl.pallas_call(
        _mlp_body,
        grid=(NPROG,),
        in_specs=[
            pl.BlockSpec((TILE, D), lambda t: (t, 0)),
            pl.BlockSpec((D, D), lambda t: (0, 0)),
            pl.BlockSpec((D,), lambda t: (0,)),
            pl.BlockSpec((D, D), lambda t: (0, 0)),
            pl.BlockSpec((D,), lambda t: (0,)),
            pl.BlockSpec((1, D), lambda t: (0, 0)),
            pl.BlockSpec((1,), lambda t: (0,)),
        ],
        out_specs=pl.BlockSpec((1, 1, TILE), lambda t: (t, 0, 0)),
        out_shape=jax.ShapeDtypeStruct((NPROG, 1, TILE), jnp.float32),
    )(edge_attr, W0, b0, W1, b1, Wout, bout)


NC = 2
NS = 16
GPC = B // NC
WORDS = B * N * N
GW = N * N
TR = 62496
TSZ = 62560
CH = 4000
NCH = E // CH


def _sc_body(addr_hbm, vals_hbm, zeros_hbm, out_hbm,
             grid, abuf0, abuf1, vbuf0, vbuf1, hits, sem0, sem1):
    c = lax.axis_index("c")
    s = lax.axis_index("s")
    lo = s * TR
    zeros16 = jnp.zeros((16,), jnp.float32)
    iota16 = lax.iota(jnp.int32, 16)
    abufs, vbufs, sems = (abuf0, abuf1), (vbuf0, vbuf1), (sem0, sem1)

    pltpu.sync_copy(zeros_hbm, grid)

    def round_body(k, _):
        g = c * GPC + k
        ebase = pl.multiple_of(g * E, 8)

        def fire(ch, slot):
            off = pl.multiple_of(ebase + ch * CH, 8)
            pltpu.async_copy(addr_hbm.at[pl.ds(off, CH)], abufs[slot],
                             sems[slot])
            pltpu.async_copy(vals_hbm.at[pl.ds(off, CH)], vbufs[slot],
                             sems[slot])

        def waitc(slot):
            pltpu.make_async_copy(
                addr_hbm.at[pl.ds(ebase, CH)], abufs[slot], sems[slot]).wait()
            pltpu.make_async_copy(
                vals_hbm.at[pl.ds(ebase, CH)], vbufs[slot], sems[slot]).wait()

        fire(0, 0)
        acc = jnp.int32(0)
        for ch in range(NCH):
            slot = ch % 2
            waitc(slot)
            if ch + 1 < NCH:
                fire(ch + 1, 1 - slot)

            def grp(q, a_cc, slot=slot):
                a = abufs[slot][pl.ds(q * 16, 16)]
                v = vbufs[slot][pl.ds(q * 16, 16)]
                rel = a - lo
                m = plsc.bitcast(rel, jnp.uint32) < jnp.uint32(TSZ)
                plsc.store_scatter(grid, [rel], v, mask=m)
                plsc.store_compressed(hits.at[pl.ds(a_cc, 16)], rel, mask=m)
                cnt = plsc.all_reduce_population_count(m)
                return a_cc + cnt[0]
            acc = lax.fori_loop(0, CH // 16, grp, acc)

        oo = pl.multiple_of(g * GW + s * TR, 8)
        pltpu.sync_copy(grid, out_hbm.at[pl.ds(oo, TSZ)])

        def cgrp(q, _):
            rel = hits[pl.ds(q * 16, 16)]
            m = (q * 16 + iota16) < acc
            plsc.store_scatter(grid, [rel], zeros16, mask=m)
            return 0
        lax.fori_loop(0, (acc + 15) // 16, cgrp, 0)
        return 0

    lax.fori_loop(0, GPC, round_body, 0)


def _run_scatter(addr1, vals1, zeros_w):
    mesh = plsc.VectorSubcoreMesh(core_axis_name="c", subcore_axis_name="s")
    kern = pl.kernel(
        _sc_body,
        out_type=jax.ShapeDtypeStruct((WORDS,), jnp.float32),
        mesh=mesh,
        compiler_params=pltpu.CompilerParams(needs_layout_passes=False),
        scratch_types=[
            pltpu.VMEM((TSZ,), jnp.float32),
            pltpu.VMEM((CH,), jnp.int32),
            pltpu.VMEM((CH,), jnp.int32),
            pltpu.VMEM((CH,), jnp.float32),
            pltpu.VMEM((CH,), jnp.float32),
            pltpu.VMEM((E,), jnp.int32),
            pltpu.SemaphoreType.DMA,
            pltpu.SemaphoreType.DMA,
        ],
    )
    return kern(addr1, vals1, zeros_w)


def _sym_body(a_ref, o_ref):
    x = a_ref[...]
    o_ref[0] = x + x.T


def _run_sym(S2):
    return pl.pallas_call(
        _sym_body,
        grid=(B,),
        in_specs=[pl.BlockSpec((N, N), lambda b: (b, 0))],
        out_specs=pl.BlockSpec((1, N, N), lambda b: (b, 0, 0)),
        out_shape=jax.ShapeDtypeStruct((B, N, N), jnp.float32),
    )(S2)


@jax.jit
def kernel(edge_attr, edge_index, W0, b0, W1, b1, Wout, bout):
    vals = _run_mlp(edge_attr, W0, b0, W1, b1, Wout, bout)
    addr = _run_addr(edge_index)
    zeros_w = jnp.zeros((TSZ,), jnp.float32)
    S = _run_scatter(addr.reshape(BE), vals.reshape(BE), zeros_w)
    return _run_sym(S.reshape(B * N, N))


# speedup vs baseline: 6.9191x; 1.0569x over previous
import functools

import jax
import jax.numpy as jnp
from jax import lax
from jax.experimental import pallas as pl
from jax.experimental.pallas import tpu as pltpu
from jax.experimental.pallas import tpu_sc as plsc

B = 16
N = 1000
E = 32000
D = 128
BE = B * E


TILE = 16000
NPROG = BE // TILE
TPG = E // TILE


def _mlp_body(x_ref, w0_ref, b0_ref, w1_ref, b1_ref, wout_ref,
              bout_ref, val_ref):
    bf = jnp.bfloat16
    w0 = w0_ref[...].astype(bf)
    w1 = w1_ref[...].astype(bf)

    def head(x):
        a1 = lax.dot_general(x, w0, (((1,), (1,)), ((), ())),
                             preferred_element_type=jnp.float32) + b0_ref[...]
        h1 = a1.astype(bf)
        h1 = h1 * lax.logistic(h1)
        a2 = lax.dot_general(h1, w1, (((1,), (1,)), ((), ())),
                             preferred_element_type=jnp.float32) + b1_ref[...]
        h2 = a2.astype(bf)
        h = (h2 * lax.logistic(h2)).astype(jnp.float32)
        return lax.dot_general(wout_ref[...], h, (((1,), (1,)), ((), ())),
                               preferred_element_type=jnp.float32)

    x = x_ref[...].astype(bf)
    v = head(x) + bout_ref[...]
    val_ref[0] = lax.logistic(v) * 5.0


def _run_mlp(edge_attr, W0, b0, W1, b1, Wout, bout):
    return pl.pallas_call(
        _mlp_body,
        grid=(NPROG,),
        in_specs=[
            pl.BlockSpec((TILE, D), lambda t: (t, 0)),
            pl.BlockSpec((D, D), lambda t: (0, 0)),
            pl.BlockSpec((D,), lambda t: (0,)),
            pl.BlockSpec((D, D), lambda t: (0, 0)),
            pl.BlockSpec((D,), lambda t: (0,)),
            pl.BlockSpec((1, D), lambda t: (0, 0)),
            pl.BlockSpec((1,), lambda t: (0,)),
        ],
        out_specs=pl.BlockSpec((1, 1, TILE), lambda t: (t, 0, 0)),
        out_shape=jax.ShapeDtypeStruct((NPROG, 1, TILE), jnp.float32),
    )(edge_attr, W0, b0, W1, b1, Wout, bout)


NC = 2
NS = 16
GPC = B // NC
GRT = 64
LROWS = N - 15 * GRT
CH = 4000
NCH = E // CH


def _sc_body(row_hbm, col_hbm, vals_hbm, zeros_hbm, out_hbm,
             grid, ib0, ib1, jb0, jb1, vb0, vb1, sem0, sem1, wsem):
    c = lax.axis_index("c")
    s = lax.axis_index("s")
    row_lo = s * GRT
    nrows = jnp.where(s < 15, GRT, LROWS).astype(jnp.uint32)
    ibufs, jbufs, vbufs = (ib0, ib1), (jb0, jb1), (vb0, vb1)
    sems = (sem0, sem1)

    def round_body(k, _):
        g = c * GPC + k
        ebase = pl.multiple_of(g * E, 8)

        @pl.when(s < 15)
        def _():
            pltpu.async_copy(zeros_hbm, grid, wsem)
        @pl.when(s == 15)
        def _():
            pltpu.async_copy(zeros_hbm.at[pl.ds(0, LROWS)],
                             grid.at[pl.ds(0, LROWS)], wsem)

        def fire(ch, slot):
            off = pl.multiple_of(ebase + ch * CH, 8)
            pltpu.async_copy(row_hbm.at[pl.ds(off, CH)], ibufs[slot],
                             sems[slot])
            pltpu.async_copy(col_hbm.at[pl.ds(off, CH)], jbufs[slot],
                             sems[slot])
            pltpu.async_copy(vals_hbm.at[pl.ds(off, CH)], vbufs[slot],
                             sems[slot])

        def waitc(slot):
            pltpu.make_async_copy(
                row_hbm.at[pl.ds(ebase, CH)], ibufs[slot], sems[slot]).wait()
            pltpu.make_async_copy(
                col_hbm.at[pl.ds(ebase, CH)], jbufs[slot], sems[slot]).wait()
            pltpu.make_async_copy(
                vals_hbm.at[pl.ds(ebase, CH)], vbufs[slot], sems[slot]).wait()

        fire(0, 0)
        @pl.when(s < 15)
        def _():
            pltpu.make_async_copy(zeros_hbm, grid, wsem).wait()
        @pl.when(s == 15)
        def _():
            pltpu.make_async_copy(zeros_hbm.at[pl.ds(0, LROWS)],
                                  grid.at[pl.ds(0, LROWS)], wsem).wait()

        for ch in range(NCH):
            slot = ch % 2
            waitc(slot)
            if ch + 1 < NCH:
                fire(ch + 1, 1 - slot)

            def grp(q, _, slot=slot):
                iv = ibufs[slot][pl.ds(q * 16, 16)]
                jv = jbufs[slot][pl.ds(q * 16, 16)]
                vv = vbufs[slot][pl.ds(q * 16, 16)]
                ri = iv - row_lo
                m = plsc.bitcast(ri, jnp.uint32) < nrows
                plsc.store_scatter(grid, [ri, jv], vv, mask=m)
                return 0
            lax.fori_loop(0, CH // 16, grp, 0)

        ro = pl.multiple_of(g * N + s * GRT, 8)
        @pl.when(s < 15)
        def _():
            pltpu.sync_copy(grid, out_hbm.at[pl.ds(ro, GRT)])
        @pl.when(s == 15)
        def _():
            pltpu.sync_copy(grid.at[pl.ds(0, LROWS)],
                            out_hbm.at[pl.ds(ro, LROWS)])
        return 0

    lax.fori_loop(0, GPC, round_body, 0)


def _run_scatter(rows1, cols1, vals1, zeros_w):
    mesh = plsc.VectorSubcoreMesh(core_axis_name="c", subcore_axis_name="s")
    kern = pl.kernel(
        _sc_body,
        out_type=jax.ShapeDtypeStruct((B * N, N), jnp.float32),
        mesh=mesh,
        compiler_params=pltpu.CompilerParams(needs_layout_passes=False),
        scratch_types=[
            pltpu.VMEM((GRT, N), jnp.float32),
            pltpu.VMEM((CH,), jnp.int32),
            pltpu.VMEM((CH,), jnp.int32),
            pltpu.VMEM((CH,), jnp.int32),
            pltpu.VMEM((CH,), jnp.int32),
            pltpu.VMEM((CH,), jnp.float32),
            pltpu.VMEM((CH,), jnp.float32),
            pltpu.SemaphoreType.DMA,
            pltpu.SemaphoreType.DMA,
            pltpu.SemaphoreType.DMA,
        ],
    )
    return kern(rows1, cols1, vals1, zeros_w)


def _sym_body(a_ref, o_ref):
    x = a_ref[...]
    o_ref[0] = x + x.T


def _run_sym(S2):
    return pl.pallas_call(
        _sym_body,
        grid=(B,),
        in_specs=[pl.BlockSpec((N, N), lambda b: (b, 0))],
        out_specs=pl.BlockSpec((1, N, N), lambda b: (b, 0, 0)),
        out_shape=jax.ShapeDtypeStruct((B, N, N), jnp.float32),
    )(S2)


@jax.jit
def kernel(edge_attr, edge_index, W0, b0, W1, b1, Wout, bout):
    vals = _run_mlp(edge_attr, W0, b0, W1, b1, Wout, bout)
    rows1 = edge_index[:, 0, :].reshape(BE)
    cols1 = edge_index[:, 1, :].reshape(BE)
    zeros_w = jnp.zeros((GRT, N), jnp.float32)
    S = _run_scatter(rows1, cols1, vals.reshape(BE), zeros_w)
    return _run_sym(S)


# speedup vs baseline: 6.9507x; 1.0046x over previous
import functools

import jax
import jax.numpy as jnp
from jax import lax
from jax.experimental import pallas as pl
from jax.experimental.pallas import tpu as pltpu
from jax.experimental.pallas import tpu_sc as plsc

B = 16
N = 1000
E = 32000
D = 128
BE = B * E


TILE = 16000
NPROG = BE // TILE
TPG = E // TILE


def _mlp_body(x_ref, w0_ref, b0_ref, w1_ref, b1_ref, wout_ref,
              bout_ref, val_ref):
    bf = jnp.bfloat16
    w0 = w0_ref[...].astype(bf)
    w1 = w1_ref[...].astype(bf)

    def head(x):
        a1 = lax.dot_general(x, w0, (((1,), (1,)), ((), ())),
                             preferred_element_type=jnp.float32) + b0_ref[...]
        h1 = a1.astype(bf)
        h1 = h1 * lax.logistic(h1)
        a2 = lax.dot_general(h1, w1, (((1,), (1,)), ((), ())),
                             preferred_element_type=jnp.float32) + b1_ref[...]
        h2 = a2.astype(bf)
        h = (h2 * lax.logistic(h2)).astype(jnp.float32)
        return lax.dot_general(wout_ref[...], h, (((1,), (1,)), ((), ())),
                               preferred_element_type=jnp.float32)

    x = x_ref[...].astype(bf)
    v = head(x) + bout_ref[...]
    val_ref[0] = lax.logistic(v) * 5.0


def _run_mlp(edge_attr, W0, b0, W1, b1, Wout, bout):
    return pl.pallas_call(
        _mlp_body,
        grid=(NPROG,),
        in_specs=[
            pl.BlockSpec((TILE, D), lambda t: (t, 0)),
            pl.BlockSpec((D, D), lambda t: (0, 0)),
            pl.BlockSpec((D,), lambda t: (0,)),
            pl.BlockSpec((D, D), lambda t: (0, 0)),
            pl.BlockSpec((D,), lambda t: (0,)),
            pl.BlockSpec((1, D), lambda t: (0, 0)),
            pl.BlockSpec((1,), lambda t: (0,)),
        ],
        out_specs=pl.BlockSpec((1, 1, TILE), lambda t: (t, 0, 0)),
        out_shape=jax.ShapeDtypeStruct((NPROG, 1, TILE), jnp.float32),
    )(edge_attr, W0, b0, W1, b1, Wout, bout)


NC = 2
NS = 16
GPC = B // NC
GRT = 64
LROWS = N - 15 * GRT
CH = 4000
NCH = E // CH


def _sc_body(row_hbm, col_hbm, vals_hbm, zeros_hbm, out_hbm,
             grid, ib0, ib1, jb0, jb1, vb0, vb1, sem0, sem1, wsem):
    c = lax.axis_index("c")
    s = lax.axis_index("s")
    row_lo = s * GRT
    nrows = jnp.where(s < 15, GRT, LROWS).astype(jnp.uint32)
    ibufs, jbufs, vbufs = (ib0, ib1), (jb0, jb1), (vb0, vb1)
    sems = (sem0, sem1)

    def round_body(k, _):
        g = c * GPC + k
        ebase = pl.multiple_of(g * E, 8)
        ro = pl.multiple_of(g * N + s * GRT, 8)

        def fire(ch, slot):
            off = pl.multiple_of(ebase + ch * CH, 8)
            pltpu.async_copy(row_hbm.at[pl.ds(off, CH)], ibufs[slot],
                             sems[slot])
            pltpu.async_copy(col_hbm.at[pl.ds(off, CH)], jbufs[slot],
                             sems[slot])
            pltpu.async_copy(vals_hbm.at[pl.ds(off, CH)], vbufs[slot],
                             sems[slot])

        def waitc(slot):
            pltpu.make_async_copy(
                row_hbm.at[pl.ds(ebase, CH)], ibufs[slot], sems[slot]).wait()
            pltpu.make_async_copy(
                col_hbm.at[pl.ds(ebase, CH)], jbufs[slot], sems[slot]).wait()
            pltpu.make_async_copy(
                vals_hbm.at[pl.ds(ebase, CH)], vbufs[slot], sems[slot]).wait()

        fire(0, 0)
        @pl.when(s < 15)
        def _():
            @pl.when(k > 0)
            def _():
                pltpu.make_async_copy(
                    grid, out_hbm.at[pl.ds(ro, GRT)], wsem).wait()
            pltpu.sync_copy(zeros_hbm, grid)
        @pl.when(s == 15)
        def _():
            @pl.when(k > 0)
            def _():
                pltpu.make_async_copy(
                    grid.at[pl.ds(0, LROWS)],
                    out_hbm.at[pl.ds(ro, LROWS)], wsem).wait()
            pltpu.sync_copy(zeros_hbm.at[pl.ds(0, LROWS)],
                            grid.at[pl.ds(0, LROWS)])

        for ch in range(NCH):
            slot = ch % 2
            waitc(slot)
            if ch + 1 < NCH:
                fire(ch + 1, 1 - slot)

            def grp(q, _, slot=slot):
                for u in range(2):
                    iv = ibufs[slot][pl.ds(q * 32 + u * 16, 16)]
                    jv = jbufs[slot][pl.ds(q * 32 + u * 16, 16)]
                    vv = vbufs[slot][pl.ds(q * 32 + u * 16, 16)]
                    ri = iv - row_lo
                    m = plsc.bitcast(ri, jnp.uint32) < nrows
                    plsc.store_scatter(grid, [ri, jv], vv, mask=m)
                return 0
            lax.fori_loop(0, CH // 32, grp, 0)

        @pl.when(s < 15)
        def _():
            pltpu.async_copy(grid, out_hbm.at[pl.ds(ro, GRT)], wsem)
        @pl.when(s == 15)
        def _():
            pltpu.async_copy(grid.at[pl.ds(0, LROWS)],
                             out_hbm.at[pl.ds(ro, LROWS)], wsem)
        return 0

    lax.fori_loop(0, GPC, round_body, 0)

    rolast = pl.multiple_of((c * GPC + GPC - 1) * N + s * GRT, 8)
    @pl.when(s < 15)
    def _():
        pltpu.make_async_copy(grid, out_hbm.at[pl.ds(rolast, GRT)],
                              wsem).wait()
    @pl.when(s == 15)
    def _():
        pltpu.make_async_copy(grid.at[pl.ds(0, LROWS)],
                              out_hbm.at[pl.ds(rolast, LROWS)], wsem).wait()


def _run_scatter(rows1, cols1, vals1, zeros_w):
    mesh = plsc.VectorSubcoreMesh(core_axis_name="c", subcore_axis_name="s")
    kern = pl.kernel(
        _sc_body,
        out_type=jax.ShapeDtypeStruct((B * N, N), jnp.float32),
        mesh=mesh,
        compiler_params=pltpu.CompilerParams(needs_layout_passes=False),
        scratch_types=[
            pltpu.VMEM((GRT, N), jnp.float32),
            pltpu.VMEM((CH,), jnp.int32),
            pltpu.VMEM((CH,), jnp.int32),
            pltpu.VMEM((CH,), jnp.int32),
            pltpu.VMEM((CH,), jnp.int32),
            pltpu.VMEM((CH,), jnp.float32),
            pltpu.VMEM((CH,), jnp.float32),
            pltpu.SemaphoreType.DMA,
            pltpu.SemaphoreType.DMA,
            pltpu.SemaphoreType.DMA,
        ],
    )
    return kern(rows1, cols1, vals1, zeros_w)


def _sym_body(a_ref, o_ref):
    x = a_ref[...]
    o_ref[0] = x + x.T


def _run_sym(S2):
    return pl.pallas_call(
        _sym_body,
        grid=(B,),
        in_specs=[pl.BlockSpec((N, N), lambda b: (b, 0))],
        out_specs=pl.BlockSpec((1, N, N), lambda b: (b, 0, 0)),
        out_shape=jax.ShapeDtypeStruct((B, N, N), jnp.float32),
    )(S2)


@jax.jit
def kernel(edge_attr, edge_index, W0, b0, W1, b1, Wout, bout):
    vals = _run_mlp(edge_attr, W0, b0, W1, b1, Wout, bout)
    rows1 = edge_index[:, 0, :].reshape(BE)
    cols1 = edge_index[:, 1, :].reshape(BE)
    zeros_w = jnp.zeros((GRT, N), jnp.float32)
    S = _run_scatter(rows1, cols1, vals.reshape(BE), zeros_w)
    return _run_sym(S)


# speedup vs baseline: 6.9674x; 1.0024x over previous
import jax
import jax.numpy as jnp
from jax import lax
from jax.experimental import pallas as pl
from jax.experimental.pallas import tpu as pltpu
from jax.experimental.pallas import tpu_sc as plsc

B = 16
N = 1000
E = 32000
D = 128
BE = B * E


TILE = 16000
NPROG = BE // TILE
TPG = E // TILE


def _mlp_body(x_ref, w0_ref, b0_ref, w1_ref, b1_ref, wout_ref,
              bout_ref, val_ref):
    bf = jnp.bfloat16
    w0 = w0_ref[...].astype(bf)
    w1 = w1_ref[...].astype(bf)

    def head(x):
        a1 = lax.dot_general(x, w0, (((1,), (1,)), ((), ())),
                             preferred_element_type=jnp.float32) + b0_ref[...]
        h1 = a1.astype(bf)
        h1 = h1 * lax.logistic(h1)
        a2 = lax.dot_general(h1, w1, (((1,), (1,)), ((), ())),
                             preferred_element_type=jnp.float32) + b1_ref[...]
        h2 = a2.astype(bf)
        h = (h2 * lax.logistic(h2)).astype(jnp.float32)
        return lax.dot_general(wout_ref[...], h, (((1,), (1,)), ((), ())),
                               preferred_element_type=jnp.float32)

    x = x_ref[...].astype(bf)
    v = head(x) + bout_ref[...]
    val_ref[0] = lax.logistic(v) * 5.0


def _run_mlp(edge_attr, W0, b0, W1, b1, Wout, bout):
    return pl.pallas_call(
        _mlp_body,
        grid=(NPROG,),
        in_specs=[
            pl.BlockSpec((TILE, D), lambda t: (t, 0)),
            pl.BlockSpec((D, D), lambda t: (0, 0)),
            pl.BlockSpec((D,), lambda t: (0,)),
            pl.BlockSpec((D, D), lambda t: (0, 0)),
            pl.BlockSpec((D,), lambda t: (0,)),
            pl.BlockSpec((1, D), lambda t: (0, 0)),
            pl.BlockSpec((1,), lambda t: (0,)),
        ],
        out_specs=pl.BlockSpec((1, 1, TILE), lambda t: (t, 0, 0)),
        out_shape=jax.ShapeDtypeStruct((NPROG, 1, TILE), jnp.float32),
    )(edge_attr, W0, b0, W1, b1, Wout, bout)


NC = 2
NS = 16
GPC = B // NC
GRT = 64
LROWS = N - 15 * GRT
CH = 4000
NCH = E // CH


def _sc_body(row_hbm, col_hbm, vals_hbm, zeros_hbm, out_hbm,
             grid, ib0, ib1, jb0, jb1, vb0, vb1, sem0, sem1, wsem):
    c = lax.axis_index("c")
    s = lax.axis_index("s")
    row_lo = s * GRT
    nrows = jnp.where(s < 15, GRT, LROWS).astype(jnp.uint32)
    ibufs, jbufs, vbufs = (ib0, ib1), (jb0, jb1), (vb0, vb1)
    sems = (sem0, sem1)

    def round_body(k, _):
        g = c * GPC + k
        ebase = pl.multiple_of(g * E, 8)
        ro = pl.multiple_of(g * N + s * GRT, 8)

        def fire(ch, slot):
            off = pl.multiple_of(ebase + ch * CH, 8)
            pltpu.async_copy(row_hbm.at[pl.ds(off, CH)], ibufs[slot],
                             sems[slot])
            pltpu.async_copy(col_hbm.at[pl.ds(off, CH)], jbufs[slot],
                             sems[slot])
            pltpu.async_copy(vals_hbm.at[pl.ds(off, CH)], vbufs[slot],
                             sems[slot])

        def waitc(slot):
            pltpu.make_async_copy(
                row_hbm.at[pl.ds(ebase, CH)], ibufs[slot], sems[slot]).wait()
            pltpu.make_async_copy(
                col_hbm.at[pl.ds(ebase, CH)], jbufs[slot], sems[slot]).wait()
            pltpu.make_async_copy(
                vals_hbm.at[pl.ds(ebase, CH)], vbufs[slot], sems[slot]).wait()

        fire(0, 0)
        @pl.when(s < 15)
        def _():
            @pl.when(k > 0)
            def _():
                pltpu.make_async_copy(
                    grid, out_hbm.at[pl.ds(ro, GRT)], wsem).wait()
            pltpu.sync_copy(zeros_hbm, grid)
        @pl.when(s == 15)
        def _():
            @pl.when(k > 0)
            def _():
                pltpu.make_async_copy(
                    grid.at[pl.ds(0, LROWS)],
                    out_hbm.at[pl.ds(ro, LROWS)], wsem).wait()
            pltpu.sync_copy(zeros_hbm.at[pl.ds(0, LROWS)],
                            grid.at[pl.ds(0, LROWS)])

        for ch in range(NCH):
            slot = ch % 2
            waitc(slot)
            if ch + 1 < NCH:
                fire(ch + 1, 1 - slot)

            def grp(q, _, slot=slot):
                for u in range(2):
                    iv = ibufs[slot][pl.ds(q * 32 + u * 16, 16)]
                    jv = jbufs[slot][pl.ds(q * 32 + u * 16, 16)]
                    vv = vbufs[slot][pl.ds(q * 32 + u * 16, 16)]
                    ri = iv - row_lo
                    m = plsc.bitcast(ri, jnp.uint32) < nrows
                    plsc.store_scatter(grid, [ri, jv], vv, mask=m)
                return 0
            lax.fori_loop(0, CH // 32, grp, 0)

        @pl.when(s < 15)
        def _():
            pltpu.async_copy(grid, out_hbm.at[pl.ds(ro, GRT)], wsem)
        @pl.when(s == 15)
        def _():
            pltpu.async_copy(grid.at[pl.ds(0, LROWS)],
                             out_hbm.at[pl.ds(ro, LROWS)], wsem)
        return 0

    lax.fori_loop(0, GPC, round_body, 0)

    rolast = pl.multiple_of((c * GPC + GPC - 1) * N + s * GRT, 8)
    @pl.when(s < 15)
    def _():
        pltpu.make_async_copy(grid, out_hbm.at[pl.ds(rolast, GRT)],
                              wsem).wait()
    @pl.when(s == 15)
    def _():
        pltpu.make_async_copy(grid.at[pl.ds(0, LROWS)],
                              out_hbm.at[pl.ds(rolast, LROWS)], wsem).wait()


def _run_scatter(rows1, cols1, vals1, zeros_w):
    mesh = plsc.VectorSubcoreMesh(core_axis_name="c", subcore_axis_name="s")
    kern = pl.kernel(
        _sc_body,
        out_type=jax.ShapeDtypeStruct((B * N, N), jnp.float32),
        mesh=mesh,
        compiler_params=pltpu.CompilerParams(needs_layout_passes=False),
        scratch_types=[
            pltpu.VMEM((GRT, N), jnp.float32),
            pltpu.VMEM((CH,), jnp.int32),
            pltpu.VMEM((CH,), jnp.int32),
            pltpu.VMEM((CH,), jnp.int32),
            pltpu.VMEM((CH,), jnp.int32),
            pltpu.VMEM((CH,), jnp.float32),
            pltpu.VMEM((CH,), jnp.float32),
            pltpu.SemaphoreType.DMA,
            pltpu.SemaphoreType.DMA,
            pltpu.SemaphoreType.DMA,
        ],
    )
    return kern(rows1, cols1, vals1, zeros_w)


def _sym_body(a_ref, o_ref):
    x = a_ref[...]
    o_ref[0] = x + x.T


def _run_sym(S2):
    return pl.pallas_call(
        _sym_body,
        grid=(B,),
        in_specs=[pl.BlockSpec((N, N), lambda b: (b, 0))],
        out_specs=pl.BlockSpec((1, N, N), lambda b: (b, 0, 0)),
        out_shape=jax.ShapeDtypeStruct((B, N, N), jnp.float32),
    )(S2)


@jax.jit
def kernel(edge_attr, edge_index, W0, b0, W1, b1, Wout, bout):
    vals = _run_mlp(edge_attr, W0, b0, W1, b1, Wout, bout)
    rows1 = edge_index[:, 0, :].reshape(BE)
    cols1 = edge_index[:, 1, :].reshape(BE)
    zeros_w = jnp.zeros((GRT, N), jnp.float32)
    S = _run_scatter(rows1, cols1, vals.reshape(BE), zeros_w)
    return _run_sym(S)
